# baseline (speedup 1.0000x reference)
import jax
import jax.numpy as jnp
from jax.experimental import pallas as pl
from jax.experimental.pallas import tpu as pltpu

_B = 16
_C = 64
_HW = 1024
_K = 1024
_T = 1024
_G = 2
_N = _B * _HW // (_T * _G)


def _vq_block_kernel(x_ref, e_ref, out_ref, loss_ref, en_ref):
    e = e_ref[...]

    @pl.when(pl.program_id(0) == 0)
    def _():
        en_ref[...] = jnp.sum(e * e, axis=1)[None, :]
        loss_ref[...] = jnp.zeros_like(loss_ref)

    e2 = e + e
    iota = jax.lax.broadcasted_iota(jnp.int32, (_T, _K), 1)

    for s in range(_G):
        x = x_ref[s].reshape(_C, _T)
        mm2 = jax.lax.dot_general(
            x, e2, (((0,), (1,)), ((), ())),
            preferred_element_type=jnp.float32)

        xnorm = jnp.sum(x * x, axis=0)
        d = (xnorm[:, None] + en_ref[...]) - mm2

        dmin = jnp.min(d, axis=1)
        idx = jnp.min(jnp.where(d == dmin[:, None], iota, _K), axis=1)

        onehot = (iota == idx[:, None]).astype(jnp.float32)
        q = jnp.dot(onehot, e, preferred_element_type=jnp.float32)

        out_ref[s] = q.T.reshape(_C, 32, 32)

        loss_ref[...] += jnp.sum(dmin).reshape(1, 1)

    @pl.when(pl.program_id(0) == _N - 1)
    def _():
        m = loss_ref[0, 0] * (1.0 / (_B * _HW * _C))
        loss_ref[...] = (m + 0.25 * m).reshape(1, 1)


def kernel(inputs, embedding):
    out, loss = pl.pallas_call(
        _vq_block_kernel,
        grid=(_N,),
        in_specs=[
            pl.BlockSpec((_G, _C, 32, 32), lambda i: (i, 0, 0, 0)),
            pl.BlockSpec((_K, _C), lambda i: (0, 0)),
        ],
        out_specs=[
            pl.BlockSpec((_G, _C, 32, 32), lambda i: (i, 0, 0, 0)),
            pl.BlockSpec((1, 1), lambda i: (0, 0)),
        ],
        out_shape=[
            jax.ShapeDtypeStruct((_B, _C, 32, 32), jnp.float32),
            jax.ShapeDtypeStruct((1, 1), jnp.float32),
        ],
        scratch_shapes=[pltpu.VMEM((1, _K), jnp.float32)],
    )(inputs, embedding)

    return out, loss.reshape(())


# speedup vs baseline: 1.3887x; 1.3887x over previous
import jax
import jax.numpy as jnp
from jax.experimental import pallas as pl
from jax.experimental.pallas import tpu as pltpu

_B = 16
_C = 64
_HW = 1024
_K = 1024
_T = 1024
_G = 2
_N = _B * _HW // (_T * _G)


def _vq_block_kernel(x_ref, e_ref, out_ref, loss_ref, en_ref):
    e = e_ref[...]

    @pl.when(pl.program_id(0) == 0)
    def _():
        en_ref[...] = jnp.sum(e * e, axis=1)[None, :]
        loss_ref[...] = jnp.zeros_like(loss_ref)

    e2 = e + e
    iota = jax.lax.broadcasted_iota(jnp.int32, (_T, _K), 1)

    for s in range(_G):
        x = x_ref[s]
        mm2 = jax.lax.dot_general(
            x, e2, (((0,), (1,)), ((), ())),
            preferred_element_type=jnp.float32)

        xnorm = jnp.sum(x * x, axis=0)
        d = (xnorm[:, None] + en_ref[...]) - mm2

        dmin = jnp.min(d, axis=1)
        idx = jnp.min(jnp.where(d == dmin[:, None], iota, _K), axis=1)

        onehot = (iota == idx[:, None]).astype(jnp.float32)
        q = jnp.dot(onehot, e, preferred_element_type=jnp.float32)

        out_ref[s] = q.T

        loss_ref[...] += jnp.sum(dmin).reshape(1, 1)

    @pl.when(pl.program_id(0) == _N - 1)
    def _():
        m = loss_ref[0, 0] * (1.0 / (_B * _HW * _C))
        loss_ref[...] = (m + 0.25 * m).reshape(1, 1)


def kernel(inputs, embedding):
    x3 = inputs.reshape(_N * _G, _C, _T)
    out, loss = pl.pallas_call(
        _vq_block_kernel,
        grid=(_N,),
        in_specs=[
            pl.BlockSpec((_G, _C, _T), lambda i: (i, 0, 0)),
            pl.BlockSpec((_K, _C), lambda i: (0, 0)),
        ],
        out_specs=[
            pl.BlockSpec((_G, _C, _T), lambda i: (i, 0, 0)),
            pl.BlockSpec((1, 1), lambda i: (0, 0)),
        ],
        out_shape=[
            jax.ShapeDtypeStruct((_N * _G, _C, _T), jnp.float32),
            jax.ShapeDtypeStruct((1, 1), jnp.float32),
        ],
        scratch_shapes=[pltpu.VMEM((1, _K), jnp.float32)],
    )(x3, embedding)

    return out.reshape(_B, _C, 32, 32), loss.reshape(())


# speedup vs baseline: 1.4195x; 1.0221x over previous
import jax
import jax.numpy as jnp
from jax.experimental import pallas as pl
from jax.experimental.pallas import tpu as pltpu

_B = 16
_C = 64
_HW = 1024
_K = 1024
_T = 1024
_G = 4
_N = _B * _HW // (_T * _G)


def _vq_block_kernel(x_ref, e_ref, out_ref, loss_ref, en_ref):
    e = e_ref[...]

    @pl.when(pl.program_id(0) == 0)
    def _():
        en_ref[...] = jnp.sum(e * e, axis=1)[None, :]
        loss_ref[...] = jnp.zeros_like(loss_ref)

    e2 = e + e
    iota = jax.lax.broadcasted_iota(jnp.int32, (_T, _K), 1)

    for s in range(_G):
        x = x_ref[s]
        mm2 = jax.lax.dot_general(
            x, e2, (((0,), (1,)), ((), ())),
            preferred_element_type=jnp.float32)

        xnorm = jnp.sum(x * x, axis=0)
        d = (xnorm[:, None] + en_ref[...]) - mm2

        dmin = jnp.min(d, axis=1)
        idx = jnp.min(jnp.where(d == dmin[:, None], iota, _K), axis=1)

        onehot = (iota == idx[:, None]).astype(jnp.float32)
        q = jnp.dot(onehot, e, preferred_element_type=jnp.float32)

        out_ref[s] = q.T

        loss_ref[...] += jnp.sum(dmin).reshape(1, 1)

    @pl.when(pl.program_id(0) == _N - 1)
    def _():
        m = loss_ref[0, 0] * (1.0 / (_B * _HW * _C))
        loss_ref[...] = (m + 0.25 * m).reshape(1, 1)


def kernel(inputs, embedding):
    x3 = inputs.reshape(_N * _G, _C, _T)
    out, loss = pl.pallas_call(
        _vq_block_kernel,
        grid=(_N,),
        in_specs=[
            pl.BlockSpec((_G, _C, _T), lambda i: (i, 0, 0)),
            pl.BlockSpec((_K, _C), lambda i: (0, 0)),
        ],
        out_specs=[
            pl.BlockSpec((_G, _C, _T), lambda i: (i, 0, 0)),
            pl.BlockSpec((1, 1), lambda i: (0, 0)),
        ],
        out_shape=[
            jax.ShapeDtypeStruct((_N * _G, _C, _T), jnp.float32),
            jax.ShapeDtypeStruct((1, 1), jnp.float32),
        ],
        scratch_shapes=[pltpu.VMEM((1, _K), jnp.float32)],
    )(x3, embedding)

    return out.reshape(_B, _C, 32, 32), loss.reshape(())
